# Initial kernel scaffold; baseline (speedup 1.0000x reference)
import functools

import jax
import jax.numpy as jnp
from jax import lax
from jax.experimental import pallas as pl
from jax.experimental.pallas import tpu as pltpu
from jax.experimental.pallas import tpu_sc as plsc

N_NODES = 10000
N_EDGES = 320000
D_FEAT = 128
HIDDEN = 128

NPAD = 10240
HALF = HIDDEN // 2
N_TILES = 16
CHUNK = 128
K_CHUNKS = 157
EDGES_PER_TILE = CHUNK * K_CHUNKS
E_PAD = EDGES_PER_TILE * N_TILES
ROWS_PER_TILE = NPAD // N_TILES


def _mm_body(x_ref, w_ref, o_ref):
    o_ref[...] = jnp.dot(x_ref[...], w_ref[...],
                         preferred_element_type=jnp.float32)


def _matmul_split(x_pad, W):
    rblk = 1280
    nrb = NPAD // rblk
    return pl.pallas_call(
        _mm_body,
        grid=(2, nrb),
        in_specs=[
            pl.BlockSpec((rblk, D_FEAT), lambda c, r: (r, 0)),
            pl.BlockSpec((D_FEAT, HALF), lambda c, r: (0, c)),
        ],
        out_specs=pl.BlockSpec((rblk, HALF), lambda c, r: (c * nrb + r, 0)),
        out_shape=jax.ShapeDtypeStruct((2 * NPAD, HALF), jnp.float32),
    )(x_pad, W)


def _sc_body(hp_hbm, src_hbm, dst_hbm, zeros_hbm, b_hbm, out_hbm,
             agg_sp, srcbuf, dstbuf, rows_v, obuf, bias_v, sem):
    c = lax.axis_index("c")
    s = lax.axis_index("s")

    pltpu.sync_copy(src_hbm.at[c, s], srcbuf)
    pltpu.sync_copy(dst_hbm.at[s], dstbuf)
    pltpu.sync_copy(zeros_hbm, agg_sp.at[pl.ds(s * ROWS_PER_TILE, ROWS_PER_TILE)])
    plsc.subcore_barrier()

    def step(j, carry):
        cp = pltpu.make_async_copy(hp_hbm.at[srcbuf.at[j]], rows_v, sem)
        cp.start()
        cp.wait()
        pltpu.sync_copy(rows_v, agg_sp.at[dstbuf.at[j]], add=True)
        return carry

    lax.fori_loop(0, K_CHUNKS, step, 0, unroll=False)
    plsc.subcore_barrier()

    pltpu.sync_copy(b_hbm.at[pl.ds(c * HALF, HALF)], bias_v)
    pltpu.sync_copy(agg_sp.at[pl.ds(s * ROWS_PER_TILE, ROWS_PER_TILE)], obuf)

    def eprow(r, carry):
        for j in range(HALF // 16):
            sl = pl.ds(j * 16, 16)
            obuf[r, sl] = jnp.maximum(obuf[r, sl] + bias_v[sl], 0.0)
        return carry

    lax.fori_loop(0, ROWS_PER_TILE, eprow, 0, unroll=False)
    pltpu.sync_copy(
        obuf,
        out_hbm.at[pl.ds(s * ROWS_PER_TILE, ROWS_PER_TILE),
                   pl.ds(c * HALF, HALF)])


@functools.partial(
    pl.kernel,
    out_type=jax.ShapeDtypeStruct((NPAD, HIDDEN), jnp.float32),
    mesh=plsc.VectorSubcoreMesh(core_axis_name="c", subcore_axis_name="s"),
    scratch_types=[
        pltpu.VMEM_SHARED((NPAD, HALF), jnp.float32),
        pltpu.VMEM((K_CHUNKS, CHUNK), jnp.int32),
        pltpu.VMEM((K_CHUNKS, CHUNK), jnp.int32),
        pltpu.VMEM((CHUNK, HALF), jnp.float32),
        pltpu.VMEM((ROWS_PER_TILE, HALF), jnp.float32),
        pltpu.VMEM((HALF,), jnp.float32),
        pltpu.SemaphoreType.DMA,
    ],
)
def _sc_aggregate(hp, src3, dst3, zeros, b, out, *scratch):
    _sc_body(hp, src3, dst3, zeros, b, out, *scratch)


def kernel(x, edge_index, W, b):
    x_pad = jnp.zeros((NPAD, D_FEAT), jnp.float32).at[:N_NODES].set(x)
    hp = _matmul_split(x_pad, W)

    src = edge_index[0].astype(jnp.int32)
    dst = edge_index[1].astype(jnp.int32)
    npad_e = E_PAD - N_EDGES
    spread = N_NODES + (jnp.arange(npad_e, dtype=jnp.int32) % (NPAD - N_NODES))
    src_p = jnp.concatenate([src, spread]).reshape(N_TILES, K_CHUNKS, CHUNK)
    dst_p = jnp.concatenate([dst, spread]).reshape(N_TILES, K_CHUNKS, CHUNK)
    src3 = jnp.stack([src_p, src_p + NPAD])
    zeros = jnp.zeros((ROWS_PER_TILE, HALF), jnp.float32)

    out_pad = _sc_aggregate(hp, src3, dst_p, zeros, b)
    return out_pad[:N_NODES]


# speedup vs baseline: 8.5199x; 8.5199x over previous
import functools

import jax
import jax.numpy as jnp
from jax import lax
from jax.experimental import pallas as pl
from jax.experimental.pallas import tpu as pltpu
from jax.experimental.pallas import tpu_sc as plsc

N_NODES = 10000
N_EDGES = 320000
D_FEAT = 128
HIDDEN = 128

NPAD = 10240
N_CORES = 2
N_TILES = 16
CHUNK = 128
K_CHUNKS = 79
EDGES_PER_WORKER = CHUNK * K_CHUNKS
E_PAD = EDGES_PER_WORKER * N_TILES * N_CORES
ROWS_PER_TILE = NPAD // N_TILES


def _mm_body(x_ref, w_ref, o_ref):
    o_ref[...] = jnp.dot(x_ref[...], w_ref[...],
                         preferred_element_type=jnp.float32)


def _matmul(x_pad, W):
    rblk = 2048
    return pl.pallas_call(
        _mm_body,
        grid=(NPAD // rblk,),
        in_specs=[
            pl.BlockSpec((rblk, D_FEAT), lambda r: (r, 0)),
            pl.BlockSpec((D_FEAT, HIDDEN), lambda r: (0, 0)),
        ],
        out_specs=pl.BlockSpec((rblk, HIDDEN), lambda r: (r, 0)),
        out_shape=jax.ShapeDtypeStruct((NPAD, HIDDEN), jnp.float32),
    )(x_pad, W)


def _sc_body(hp_hbm, src_hbm, dst_hbm, zeros_hbm, out_hbm,
             agg_sp, srcbuf, dstbuf, rows_v, sem):
    c = lax.axis_index("c")
    s = lax.axis_index("s")

    pltpu.sync_copy(src_hbm.at[c, s], srcbuf)
    pltpu.sync_copy(dst_hbm.at[c, s], dstbuf)
    rows = pl.ds(s * ROWS_PER_TILE, ROWS_PER_TILE)
    pltpu.sync_copy(zeros_hbm, agg_sp.at[rows])
    plsc.subcore_barrier()

    def step(j, carry):
        cp = pltpu.make_async_copy(hp_hbm.at[srcbuf.at[j]], rows_v, sem)
        cp.start()
        cp.wait()
        pltpu.sync_copy(rows_v, agg_sp.at[dstbuf.at[j]], add=True)
        return carry

    lax.fori_loop(0, K_CHUNKS, step, 0, unroll=False)
    plsc.subcore_barrier()

    pltpu.sync_copy(agg_sp.at[rows], out_hbm.at[c, rows])


@functools.cache
def _sc_aggregate():
    @functools.partial(
        pl.kernel,
        out_type=jax.ShapeDtypeStruct((N_CORES, NPAD, HIDDEN), jnp.float32),
        mesh=plsc.VectorSubcoreMesh(core_axis_name="c", subcore_axis_name="s"),
        scratch_types=[
            pltpu.VMEM_SHARED((NPAD, HIDDEN), jnp.float32),
            pltpu.VMEM((K_CHUNKS, CHUNK), jnp.int32),
            pltpu.VMEM((K_CHUNKS, CHUNK), jnp.int32),
            pltpu.VMEM((CHUNK, HIDDEN), jnp.float32),
            pltpu.SemaphoreType.DMA,
        ],
    )
    def agg(hp, src4, dst4, zeros, out, *scratch):
        _sc_body(hp, src4, dst4, zeros, out, *scratch)

    return agg


def _combine_body(p_ref, b_ref, o_ref):
    o_ref[...] = jnp.maximum(p_ref[0] + p_ref[1] + b_ref[...], 0.0)


def _combine(parts, b2):
    rblk = 2000
    return pl.pallas_call(
        _combine_body,
        grid=(N_NODES // rblk,),
        in_specs=[
            pl.BlockSpec((N_CORES, rblk, HIDDEN), lambda r: (0, r, 0)),
            pl.BlockSpec((1, HIDDEN), lambda r: (0, 0)),
        ],
        out_specs=pl.BlockSpec((rblk, HIDDEN), lambda r: (r, 0)),
        out_shape=jax.ShapeDtypeStruct((N_NODES, HIDDEN), jnp.float32),
    )(parts, b2)


def kernel(x, edge_index, W, b):
    x_pad = jnp.zeros((NPAD, D_FEAT), jnp.float32).at[:N_NODES].set(x)
    hp = _matmul(x_pad, W)

    src = edge_index[0].astype(jnp.int32)
    dst = edge_index[1].astype(jnp.int32)
    npad_e = E_PAD - N_EDGES
    spread = N_NODES + (jnp.arange(npad_e, dtype=jnp.int32) % (NPAD - N_NODES))
    shard = (N_CORES, N_TILES, K_CHUNKS, CHUNK)
    src4 = jnp.concatenate([src, spread]).reshape(shard)
    dst4 = jnp.concatenate([dst, spread]).reshape(shard)
    zeros = jnp.zeros((ROWS_PER_TILE, HIDDEN), jnp.float32)

    parts = _sc_aggregate()(hp, src4, dst4, zeros)
    return _combine(parts, b.reshape(1, HIDDEN))


# speedup vs baseline: 12.1116x; 1.4216x over previous
import functools

import jax
import jax.numpy as jnp
from jax import lax
from jax.experimental import pallas as pl
from jax.experimental.pallas import tpu as pltpu
from jax.experimental.pallas import tpu_sc as plsc

N_NODES = 10000
N_EDGES = 320000
D_FEAT = 128
HIDDEN = 128

NPAD = 10240
N_CORES = 2
N_TILES = 16
CHUNK = 128
K_CHUNKS = 80
EDGES_PER_WORKER = CHUNK * K_CHUNKS
E_PAD = EDGES_PER_WORKER * N_TILES * N_CORES
ROWS_PER_TILE = NPAD // N_TILES


def _mm_body(x_ref, w_ref, o_ref):
    o_ref[...] = jnp.dot(x_ref[...], w_ref[...],
                         preferred_element_type=jnp.float32)


def _matmul(x_pad, W):
    rblk = 2048
    return pl.pallas_call(
        _mm_body,
        grid=(NPAD // rblk,),
        in_specs=[
            pl.BlockSpec((rblk, D_FEAT), lambda r: (r, 0)),
            pl.BlockSpec((D_FEAT, HIDDEN), lambda r: (0, 0)),
        ],
        out_specs=pl.BlockSpec((rblk, HIDDEN), lambda r: (r, 0)),
        out_shape=jax.ShapeDtypeStruct((NPAD, HIDDEN), jnp.float32),
    )(x_pad, W)


W_CHUNKS = 16
N_WINDOWS = K_CHUNKS // W_CHUNKS


def _sc_body(hp_hbm, src_hbm, dst_hbm, zeros_hbm, out_hbm,
             agg_sp, sw0, sw1, dw0, dw1, rows_a, rows_b, sem_a, sem_b, sem_i):
    c = lax.axis_index("c")
    s = lax.axis_index("s")

    def idx_win(arr, w):
        return arr.at[c, s, pl.ds(w * W_CHUNKS, W_CHUNKS)]

    rows = pl.ds(s * ROWS_PER_TILE, ROWS_PER_TILE)
    pltpu.sync_copy(zeros_hbm, agg_sp.at[rows])
    pltpu.sync_copy(idx_win(src_hbm, 0), sw0)
    pltpu.sync_copy(idx_win(dst_hbm, 0), dw0)
    if N_WINDOWS > 1:
        pltpu.make_async_copy(idx_win(src_hbm, 1), sw1, sem_i).start()
        pltpu.make_async_copy(idx_win(dst_hbm, 1), dw1, sem_i).start()
    plsc.subcore_barrier()

    def gather(sref, jj, buf, sem):
        pltpu.make_async_copy(hp_hbm.at[sref.at[jj]], buf, sem).start()

    def finish(sref, dref, jj, buf, sem):
        pltpu.make_async_copy(hp_hbm.at[sref.at[jj]], buf, sem).wait()
        pltpu.sync_copy(buf, agg_sp.at[dref.at[jj]], add=True)

    gather(sw0, 0, rows_a, sem_a)
    gather(sw0, 1, rows_b, sem_b)
    wins = [(sw0, dw0), (sw1, dw1)]
    for w in range(N_WINDOWS):
        S, D = wins[w % 2]
        Sn, Dn = wins[(w + 1) % 2]

        def pair(i, carry, S=S, D=D):
            finish(S, D, 2 * i, rows_a, sem_a)
            gather(S, 2 * i + 2, rows_a, sem_a)
            finish(S, D, 2 * i + 1, rows_b, sem_b)
            gather(S, 2 * i + 3, rows_b, sem_b)
            return carry

        lax.fori_loop(0, W_CHUNKS // 2 - 1, pair, 0, unroll=False)
        finish(S, D, W_CHUNKS - 2, rows_a, sem_a)
        if w + 1 < N_WINDOWS:
            pltpu.make_async_copy(idx_win(src_hbm, w + 1), Sn, sem_i).wait()
            pltpu.make_async_copy(idx_win(dst_hbm, w + 1), Dn, sem_i).wait()
            gather(Sn, 0, rows_a, sem_a)
        finish(S, D, W_CHUNKS - 1, rows_b, sem_b)
        if w + 1 < N_WINDOWS:
            gather(Sn, 1, rows_b, sem_b)
        if w + 2 < N_WINDOWS:
            pltpu.make_async_copy(idx_win(src_hbm, w + 2), S, sem_i).start()
            pltpu.make_async_copy(idx_win(dst_hbm, w + 2), D, sem_i).start()
    plsc.subcore_barrier()

    pltpu.sync_copy(agg_sp.at[rows], out_hbm.at[c, rows])


@functools.cache
def _sc_aggregate():
    @functools.partial(
        pl.kernel,
        out_type=jax.ShapeDtypeStruct((N_CORES, NPAD, HIDDEN), jnp.float32),
        mesh=plsc.VectorSubcoreMesh(core_axis_name="c", subcore_axis_name="s"),
        scratch_types=[
            pltpu.VMEM_SHARED((NPAD, HIDDEN), jnp.float32),
            pltpu.VMEM((W_CHUNKS, CHUNK), jnp.int32),
            pltpu.VMEM((W_CHUNKS, CHUNK), jnp.int32),
            pltpu.VMEM((W_CHUNKS, CHUNK), jnp.int32),
            pltpu.VMEM((W_CHUNKS, CHUNK), jnp.int32),
            pltpu.VMEM((CHUNK, HIDDEN), jnp.float32),
            pltpu.VMEM((CHUNK, HIDDEN), jnp.float32),
            pltpu.SemaphoreType.DMA,
            pltpu.SemaphoreType.DMA,
            pltpu.SemaphoreType.DMA,
        ],
    )
    def agg(hp, src4, dst4, zeros, out, *scratch):
        _sc_body(hp, src4, dst4, zeros, out, *scratch)

    return agg


def _combine_body(p_ref, b_ref, o_ref):
    o_ref[...] = jnp.maximum(p_ref[0] + p_ref[1] + b_ref[...], 0.0)


def _combine(parts, b2):
    rblk = 2000
    return pl.pallas_call(
        _combine_body,
        grid=(N_NODES // rblk,),
        in_specs=[
            pl.BlockSpec((N_CORES, rblk, HIDDEN), lambda r: (0, r, 0)),
            pl.BlockSpec((1, HIDDEN), lambda r: (0, 0)),
        ],
        out_specs=pl.BlockSpec((rblk, HIDDEN), lambda r: (r, 0)),
        out_shape=jax.ShapeDtypeStruct((N_NODES, HIDDEN), jnp.float32),
    )(parts, b2)


def kernel(x, edge_index, W, b):
    x_pad = jnp.zeros((NPAD, D_FEAT), jnp.float32).at[:N_NODES].set(x)
    hp = _matmul(x_pad, W)

    src = edge_index[0].astype(jnp.int32)
    dst = edge_index[1].astype(jnp.int32)
    npad_e = E_PAD - N_EDGES
    spread = N_NODES + (jnp.arange(npad_e, dtype=jnp.int32) % (NPAD - N_NODES))
    shard = (N_CORES, N_TILES, K_CHUNKS, CHUNK)
    src4 = jnp.concatenate([src, spread]).reshape(shard)
    dst4 = jnp.concatenate([dst, spread]).reshape(shard)
    zeros = jnp.zeros((ROWS_PER_TILE, HIDDEN), jnp.float32)

    parts = _sc_aggregate()(hp, src4, dst4, zeros)
    return _combine(parts, b.reshape(1, HIDDEN))


# speedup vs baseline: 12.9409x; 1.0685x over previous
import functools

import jax
import jax.numpy as jnp
from jax import lax
from jax.experimental import pallas as pl
from jax.experimental.pallas import tpu as pltpu
from jax.experimental.pallas import tpu_sc as plsc

N_NODES = 10000
N_EDGES = 320000
D_FEAT = 128
HIDDEN = 128

NPAD = 10240
N_CORES = 2
N_TILES = 16
CHUNK = 128
K_CHUNKS = 80
EDGES_PER_WORKER = CHUNK * K_CHUNKS
E_PAD = EDGES_PER_WORKER * N_TILES * N_CORES
ROWS_PER_TILE = NPAD // N_TILES
W_CHUNKS = 16
N_WINDOWS = K_CHUNKS // W_CHUNKS


def _sc_body(x_hbm, src_hbm, dst_hbm, zeros_hbm, out_hbm,
             agg_sp, sw0, sw1, dw0, dw1, rows_a, rows_b, sem_a, sem_b, sem_i):
    c = lax.axis_index("c")
    s = lax.axis_index("s")

    def idx_win(arr, w):
        return arr.at[c, s, pl.ds(w * W_CHUNKS, W_CHUNKS)]

    rows = pl.ds(s * ROWS_PER_TILE, ROWS_PER_TILE)
    pltpu.sync_copy(zeros_hbm, agg_sp.at[rows])
    pltpu.sync_copy(idx_win(src_hbm, 0), sw0)
    pltpu.sync_copy(idx_win(dst_hbm, 0), dw0)
    if N_WINDOWS > 1:
        pltpu.make_async_copy(idx_win(src_hbm, 1), sw1, sem_i).start()
        pltpu.make_async_copy(idx_win(dst_hbm, 1), dw1, sem_i).start()
    plsc.subcore_barrier()

    def gather(sref, jj, buf, sem):
        pltpu.make_async_copy(x_hbm.at[sref.at[jj]], buf, sem).start()

    def finish(sref, dref, jj, buf, sem):
        pltpu.make_async_copy(x_hbm.at[sref.at[jj]], buf, sem).wait()
        pltpu.sync_copy(buf, agg_sp.at[dref.at[jj]], add=True)

    gather(sw0, 0, rows_a, sem_a)
    gather(sw0, 1, rows_b, sem_b)
    wins = [(sw0, dw0), (sw1, dw1)]
    for w in range(N_WINDOWS):
        S, D = wins[w % 2]
        Sn, Dn = wins[(w + 1) % 2]

        def pair(i, carry, S=S, D=D):
            finish(S, D, 2 * i, rows_a, sem_a)
            gather(S, 2 * i + 2, rows_a, sem_a)
            finish(S, D, 2 * i + 1, rows_b, sem_b)
            gather(S, 2 * i + 3, rows_b, sem_b)
            return carry

        lax.fori_loop(0, W_CHUNKS // 2 - 1, pair, 0, unroll=False)
        finish(S, D, W_CHUNKS - 2, rows_a, sem_a)
        if w + 1 < N_WINDOWS:
            pltpu.make_async_copy(idx_win(src_hbm, w + 1), Sn, sem_i).wait()
            pltpu.make_async_copy(idx_win(dst_hbm, w + 1), Dn, sem_i).wait()
            gather(Sn, 0, rows_a, sem_a)
        finish(S, D, W_CHUNKS - 1, rows_b, sem_b)
        if w + 1 < N_WINDOWS:
            gather(Sn, 1, rows_b, sem_b)
        if w + 2 < N_WINDOWS:
            pltpu.make_async_copy(idx_win(src_hbm, w + 2), S, sem_i).start()
            pltpu.make_async_copy(idx_win(dst_hbm, w + 2), D, sem_i).start()
    plsc.subcore_barrier()

    pltpu.sync_copy(agg_sp.at[rows], out_hbm.at[c, rows])


@functools.cache
def _sc_aggregate():
    @functools.partial(
        pl.kernel,
        out_type=jax.ShapeDtypeStruct((N_CORES, NPAD, D_FEAT), jnp.float32),
        mesh=plsc.VectorSubcoreMesh(core_axis_name="c", subcore_axis_name="s"),
        scratch_types=[
            pltpu.VMEM_SHARED((NPAD, D_FEAT), jnp.float32),
            pltpu.VMEM((W_CHUNKS, CHUNK), jnp.int32),
            pltpu.VMEM((W_CHUNKS, CHUNK), jnp.int32),
            pltpu.VMEM((W_CHUNKS, CHUNK), jnp.int32),
            pltpu.VMEM((W_CHUNKS, CHUNK), jnp.int32),
            pltpu.VMEM((CHUNK, D_FEAT), jnp.float32),
            pltpu.VMEM((CHUNK, D_FEAT), jnp.float32),
            pltpu.SemaphoreType.DMA,
            pltpu.SemaphoreType.DMA,
            pltpu.SemaphoreType.DMA,
        ],
    )
    def agg(x, src4, dst4, zeros, out, *scratch):
        _sc_body(x, src4, dst4, zeros, out, *scratch)

    return agg


def _mm_body(p_ref, w_ref, b_ref, o_ref):
    acc = p_ref[0] + p_ref[1]
    o_ref[...] = jnp.maximum(
        jnp.dot(acc, w_ref[...], preferred_element_type=jnp.float32)
        + b_ref[...], 0.0)


def _matmul_combine(parts, W, b2):
    rblk = 2000
    return pl.pallas_call(
        _mm_body,
        grid=(N_NODES // rblk,),
        in_specs=[
            pl.BlockSpec((N_CORES, rblk, D_FEAT), lambda r: (0, r, 0)),
            pl.BlockSpec((D_FEAT, HIDDEN), lambda r: (0, 0)),
            pl.BlockSpec((1, HIDDEN), lambda r: (0, 0)),
        ],
        out_specs=pl.BlockSpec((rblk, HIDDEN), lambda r: (r, 0)),
        out_shape=jax.ShapeDtypeStruct((N_NODES, HIDDEN), jnp.float32),
    )(parts, W, b2)


def kernel(x, edge_index, W, b):
    src = edge_index[0].astype(jnp.int32)
    dst = edge_index[1].astype(jnp.int32)
    npad_e = E_PAD - N_EDGES
    pad_src = jnp.arange(npad_e, dtype=jnp.int32) % N_NODES
    pad_dst = N_NODES + (jnp.arange(npad_e, dtype=jnp.int32) % (NPAD - N_NODES))
    shard = (N_CORES, N_TILES, K_CHUNKS, CHUNK)
    src4 = jnp.concatenate([src, pad_src]).reshape(shard)
    dst4 = jnp.concatenate([dst, pad_dst]).reshape(shard)
    zeros = jnp.zeros((ROWS_PER_TILE, D_FEAT), jnp.float32)

    parts = _sc_aggregate()(x, src4, dst4, zeros)
    return _matmul_combine(parts, W, b.reshape(1, HIDDEN))
